# baseline (speedup 1.0000x reference)
import functools

import jax
import jax.numpy as jnp
from jax.experimental import pallas as pl
from jax.experimental.pallas import tpu as pltpu
from jax.experimental.pallas import tpu_sc as plsc

H = 16
DH = 64
EPS = 1e-5
C = 1024
F = 2048
E = 8
TR = 256
RT = 256
NTE = 128
BF = jnp.bfloat16
F32 = jnp.float32


def _ln_f32(x, s, b):
    m = jnp.mean(x, axis=-1, keepdims=True)
    d = x - m
    v = jnp.mean(d * d, axis=-1, keepdims=True)
    return d * jax.lax.rsqrt(v + EPS) * s + b


def _dot_t(a, w):
    return jax.lax.dot_general(a, w, (((1,), (1,)), ((), ())),
                               preferred_element_type=F32)


def _k1_body(x_ref, s_ref, b_ref, w_ref, q_ref, k_ref, v_ref):
    h = _ln_f32(x_ref[...], s_ref[...], b_ref[...])
    qkv = _dot_t(h, w_ref[...]).astype(BF)
    for hd in range(H):
        q_ref[hd] = qkv[:, hd * DH:(hd + 1) * DH]
        k_ref[hd] = qkv[:, C + hd * DH:C + (hd + 1) * DH]
        v_ref[hd] = qkv[:, 2 * C + hd * DH:2 * C + (hd + 1) * DH]


def _ln_qkv(x, s, b, w, n):
    return pl.pallas_call(
        _k1_body,
        grid=(n // RT,),
        in_specs=[
            pl.BlockSpec((RT, C), lambda i: (i, 0)),
            pl.BlockSpec((1, C), lambda i: (0, 0)),
            pl.BlockSpec((1, C), lambda i: (0, 0)),
            pl.BlockSpec((3 * C, C), lambda i: (0, 0)),
        ],
        out_specs=[
            pl.BlockSpec((H, RT, DH), lambda i: (0, i, 0)),
            pl.BlockSpec((H, RT, DH), lambda i: (0, i, 0)),
            pl.BlockSpec((H, RT, DH), lambda i: (0, i, 0)),
        ],
        out_shape=[
            jax.ShapeDtypeStruct((H, n, DH), BF),
            jax.ShapeDtypeStruct((H, n, DH), BF),
            jax.ShapeDtypeStruct((H, n, DH), BF),
        ],
    )(x, s, b, w)


def _k2_body(q_ref, k_ref, v_ref, o_ref):
    q = (q_ref[0].astype(F32) * (DH ** -0.5)).astype(BF)
    k = k_ref[0]
    v = v_ref[0]
    s = jax.lax.dot_general(q, k, (((1,), (1,)), ((), ())),
                            preferred_element_type=F32).astype(BF)
    p = jnp.exp(s)
    l = jnp.sum(p.astype(F32), axis=-1, keepdims=True)
    o = jnp.dot(p, v, preferred_element_type=F32) / l
    o_ref[0] = o.astype(BF)


def _attention(q, k, v, n):
    return pl.pallas_call(
        _k2_body,
        grid=(H, n // RT),
        in_specs=[
            pl.BlockSpec((1, RT, DH), lambda h, i: (h, i, 0)),
            pl.BlockSpec((1, n, DH), lambda h, i: (h, 0, 0)),
            pl.BlockSpec((1, n, DH), lambda h, i: (h, 0, 0)),
        ],
        out_specs=pl.BlockSpec((1, RT, DH), lambda h, i: (h, i, 0)),
        out_shape=jax.ShapeDtypeStruct((H, n, DH), BF),
    )(q, k, v)


def _k3_body(x_ref, o_ref, w_ref, b_ref, out_ref):
    o = jnp.concatenate([o_ref[hd] for hd in range(H)], axis=1).astype(F32)
    out_ref[...] = x_ref[...] + _dot_t(o, w_ref[...]) + b_ref[...]


def _proj_res(x, o, w, b, n):
    return pl.pallas_call(
        _k3_body,
        grid=(n // RT,),
        in_specs=[
            pl.BlockSpec((RT, C), lambda i: (i, 0)),
            pl.BlockSpec((H, RT, DH), lambda i: (0, i, 0)),
            pl.BlockSpec((C, C), lambda i: (0, 0)),
            pl.BlockSpec((1, C), lambda i: (0, 0)),
        ],
        out_specs=pl.BlockSpec((RT, C), lambda i: (i, 0)),
        out_shape=jax.ShapeDtypeStruct((n, C), F32),
    )(x, o, w, b)


def _k4_body(x_ref, s_ref, b_ref, w1_ref, b1_ref, w2_ref, b2_ref, out_ref):
    x = x_ref[...]
    h = _ln_f32(x, s_ref[...], b_ref[...])
    a = _dot_t(h, w1_ref[...]) + b1_ref[...]
    g = a * 0.5 * (1.0 + jax.lax.erf(a * (2.0 ** -0.5)))
    out_ref[...] = x + _dot_t(g, w2_ref[...]) + b2_ref[...]


def _ffn(x, s, b, w1, b1, w2, b2, n):
    return pl.pallas_call(
        _k4_body,
        grid=(n // RT,),
        in_specs=[
            pl.BlockSpec((RT, C), lambda i: (i, 0)),
            pl.BlockSpec((1, C), lambda i: (0, 0)),
            pl.BlockSpec((1, C), lambda i: (0, 0)),
            pl.BlockSpec((F, C), lambda i: (0, 0)),
            pl.BlockSpec((1, F), lambda i: (0, 0)),
            pl.BlockSpec((C, F), lambda i: (0, 0)),
            pl.BlockSpec((1, C), lambda i: (0, 0)),
        ],
        out_specs=pl.BlockSpec((RT, C), lambda i: (i, 0)),
        out_shape=jax.ShapeDtypeStruct((n, C), F32),
    )(x, s, b, w1, b1, w2, b2)


def _k5_body(x_ref, s_ref, b_ref, gw_ref, h_ref, tw_ref, ti_ref):
    h = _ln_f32(x_ref[...], s_ref[...], b_ref[...])
    h_ref[...] = h.astype(BF)
    sc = jax.lax.dot_general(h, gw_ref[...], (((1,), (1,)), ((), ())),
                             precision=jax.lax.Precision.HIGHEST,
                             preferred_element_type=F32)
    sc = sc - jnp.max(sc, axis=-1, keepdims=True)
    ex = jnp.exp(sc)
    p = ex / jnp.sum(ex, axis=-1, keepdims=True)
    idx = jax.lax.broadcasted_iota(jnp.int32, (RT, E), 1)
    m1 = jnp.max(p, axis=-1, keepdims=True)
    i1 = jnp.min(jnp.where(p == m1, idx, E), axis=-1, keepdims=True)
    pm = jnp.where(idx == i1, -1.0, p)
    m2 = jnp.max(pm, axis=-1, keepdims=True)
    i2 = jnp.min(jnp.where(pm == m2, idx, E), axis=-1, keepdims=True)
    tw_ref[...] = jnp.concatenate([m1, m2], axis=1)
    ti_ref[...] = jnp.concatenate([i1, i2], axis=1).T


def _ln_gate(x, s, b, gate_w, n):
    return pl.pallas_call(
        _k5_body,
        grid=(n // RT,),
        in_specs=[
            pl.BlockSpec((RT, C), lambda i: (i, 0)),
            pl.BlockSpec((1, C), lambda i: (0, 0)),
            pl.BlockSpec((1, C), lambda i: (0, 0)),
            pl.BlockSpec((E, C), lambda i: (0, 0)),
        ],
        out_specs=[
            pl.BlockSpec((RT, C), lambda i: (i, 0)),
            pl.BlockSpec((RT, 2), lambda i: (i, 0)),
            pl.BlockSpec((2, RT), lambda i: (0, i)),
        ],
        out_shape=[
            jax.ShapeDtypeStruct((n, C), BF),
            jax.ShapeDtypeStruct((n, 2), F32),
            jax.ShapeDtypeStruct((2, n), jnp.int32),
        ],
    )(x, s, b, gate_w)


def _shift_lanes(x, sh):
    z = jnp.zeros((x.shape[0], sh), x.dtype)
    return jnp.concatenate([z, x[:, :x.shape[1] - sh]], axis=1)


def _router_body(ti_ref, p0_ref, p1_ref, te_ref):
    n = ti_ref.shape[1]
    ti = ti_ref[...]
    masks = []
    counts = []
    for e in range(E):
        m = (ti == e).astype(jnp.int32)
        masks.append(m)
        counts.append(jnp.sum(m))
    r0 = jnp.zeros((1, n), jnp.int32)
    r1 = jnp.zeros((1, n), jnp.int32)
    te = jnp.zeros((1, NTE), jnp.int32)
    tile_base = jax.lax.broadcasted_iota(jnp.int32, (1, NTE), 1) * TR
    poff = jnp.zeros((), jnp.int32)
    for e in range(E):
        m = masks[e]
        pre = m
        sh = 1
        while sh < n:
            pre = pre + _shift_lanes(pre, sh)
            sh *= 2
        t0 = pre[0:1, n - 1:n]
        rank0 = pre[0:1] - 1
        rank1 = pre[1:2] + t0 - 1
        r0 = r0 + m[0:1] * (poff + rank0)
        r1 = r1 + m[1:2] * (poff + rank1)
        padded = ((counts[e] + TR - 1) // TR) * TR
        pend = poff + padded
        te = te + (tile_base >= pend).astype(jnp.int32)
        poff = pend
    p0_ref[...] = r0
    p1_ref[...] = r1
    te_ref[...] = jnp.minimum(te, E - 1)


def _router(ti_t, n):
    return pl.pallas_call(
        _router_body,
        grid=(1,),
        in_specs=[pl.BlockSpec((2, n), lambda i: (0, 0))],
        out_specs=[
            pl.BlockSpec((1, n), lambda i: (0, 0)),
            pl.BlockSpec((1, n), lambda i: (0, 0)),
            pl.BlockSpec((1, NTE), lambda i: (0, 0)),
        ],
        out_shape=[
            jax.ShapeDtypeStruct((1, n), jnp.int32),
            jax.ShapeDtypeStruct((1, n), jnp.int32),
            jax.ShapeDtypeStruct((1, NTE), jnp.int32),
        ],
    )(ti_t)


def _k6a_body(te_ref, h_ref, p0_ref, p1_ref, w1_ref, b1_ref, w3_ref, b3_ref,
              hh_ref):
    t = pl.program_id(0)
    n = h_ref.shape[0]
    rowid = t * TR + jax.lax.broadcasted_iota(jnp.int32, (TR, n), 0)
    sel = jnp.logical_or(p0_ref[...] == rowid, p1_ref[...] == rowid)
    P = sel.astype(BF)
    xg = jnp.dot(P, h_ref[...], preferred_element_type=F32)
    a = _dot_t(xg, w1_ref[0]) + b1_ref[0]
    g = a * jax.nn.sigmoid(a)
    c = _dot_t(xg, w3_ref[0]) + b3_ref[0]
    hh_ref[...] = (g * c).astype(BF)


def _moe_up(te, h, pos0, pos1, w1, b1, w3, b3, n, nrows):
    grid_spec = pltpu.PrefetchScalarGridSpec(
        num_scalar_prefetch=1,
        grid=(nrows // TR,),
        in_specs=[
            pl.BlockSpec((n, C), lambda t, te_ref: (0, 0)),
            pl.BlockSpec((1, n), lambda t, te_ref: (0, 0)),
            pl.BlockSpec((1, n), lambda t, te_ref: (0, 0)),
            pl.BlockSpec((1, F, C), lambda t, te_ref: (te_ref[t], 0, 0)),
            pl.BlockSpec((1, 1, F), lambda t, te_ref: (te_ref[t], 0, 0)),
            pl.BlockSpec((1, F, C), lambda t, te_ref: (te_ref[t], 0, 0)),
            pl.BlockSpec((1, 1, F), lambda t, te_ref: (te_ref[t], 0, 0)),
        ],
        out_specs=pl.BlockSpec((TR, F), lambda t, te_ref: (t, 0)),
    )
    return pl.pallas_call(
        _k6a_body,
        grid_spec=grid_spec,
        out_shape=jax.ShapeDtypeStruct((nrows, F), BF),
    )(te, h, pos0, pos1, w1, b1, w3, b3)


def _k6b_body(te_ref, hh_ref, w2_ref, b2_ref, y_ref):
    hh = hh_ref[...].astype(F32)
    y_ref[...] = _dot_t(hh, w2_ref[0]) + b2_ref[0]


def _moe_down(te, hh, w2, b2, nrows):
    grid_spec = pltpu.PrefetchScalarGridSpec(
        num_scalar_prefetch=1,
        grid=(nrows // TR,),
        in_specs=[
            pl.BlockSpec((TR, F), lambda t, te_ref: (t, 0)),
            pl.BlockSpec((1, C, F), lambda t, te_ref: (te_ref[t], 0, 0)),
            pl.BlockSpec((1, 1, C), lambda t, te_ref: (te_ref[t], 0, 0)),
        ],
        out_specs=pl.BlockSpec((TR, C), lambda t, te_ref: (t, 0)),
    )
    return pl.pallas_call(
        _k6b_body,
        grid_spec=grid_spec,
        out_shape=jax.ShapeDtypeStruct((nrows, C), F32),
    )(te, hh, w2, b2)


def _sc_gather(y, p0, p1, n, nrows):
    info = plsc.get_sparse_core_info()
    nc, ns = info.num_cores, info.num_subcores
    bpw = n // (nc * ns)
    mesh = plsc.VectorSubcoreMesh(core_axis_name="c", subcore_axis_name="s")

    @functools.partial(
        pl.kernel, mesh=mesh,
        out_type=jax.ShapeDtypeStruct((2, n, C), F32),
        scratch_types=[
            pltpu.VMEM((bpw,), jnp.int32),
            pltpu.VMEM((bpw, C), F32),
            pltpu.SemaphoreType.DMA,
        ],
    )
    def k(y_hbm, p0_hbm, p1_hbm, out_hbm, idx_v, rows_v, sem):
        wid = jax.lax.axis_index("s") * nc + jax.lax.axis_index("c")
        base = wid * bpw
        pltpu.sync_copy(p0_hbm.at[pl.ds(base, bpw)], idx_v)
        pltpu.async_copy(y_hbm.at[idx_v], rows_v, sem).wait()
        pltpu.sync_copy(rows_v, out_hbm.at[0, pl.ds(base, bpw)])
        pltpu.sync_copy(p1_hbm.at[pl.ds(base, bpw)], idx_v)
        pltpu.async_copy(y_hbm.at[idx_v], rows_v, sem).wait()
        pltpu.sync_copy(rows_v, out_hbm.at[1, pl.ds(base, bpw)])

    return k(y, p0, p1)


def _k7_body(x_ref, tw_ref, g_ref, out_ref):
    out_ref[...] = (x_ref[...]
                    + tw_ref[:, 0:1] * g_ref[0]
                    + tw_ref[:, 1:2] * g_ref[1])


def _combine(pos0, pos1, x, tw, y, n, nrows):
    yg = _sc_gather(y, pos0, pos1, n, nrows)
    return pl.pallas_call(
        _k7_body,
        grid=(n // RT,),
        in_specs=[
            pl.BlockSpec((RT, C), lambda t: (t, 0)),
            pl.BlockSpec((RT, 2), lambda t: (t, 0)),
            pl.BlockSpec((2, RT, C), lambda t: (0, t, 0)),
        ],
        out_specs=pl.BlockSpec((RT, C), lambda t: (t, 0)),
        out_shape=jax.ShapeDtypeStruct((n, C), F32),
    )(x, tw, yg)


def _attn_block(x, ln_s, ln_b, qkv_w, proj_w, proj_b, n):
    q, k, v = _ln_qkv(x, ln_s, ln_b, qkv_w, n)
    o = _attention(q, k, v, n)
    return _proj_res(x, o, proj_w, proj_b.reshape(1, C), n)


def kernel(x, ln1a_s, ln1a_b, qkv_a, proj_a_w, proj_a_b, ln2a_s, ln2a_b,
           ff_w1, ff_b1, ff_w2, ff_b2, ln1b_s, ln1b_b, qkv_b, proj_b_w,
           proj_b_b, ln2b_s, ln2b_b, gate_w, e_w1, e_b1, e_w2, e_b2, e_w3,
           e_b3):
    B, N, _ = x.shape
    n = B * N
    x2 = x.reshape(n, C)
    r2 = lambda a: a.reshape(1, -1)

    x2 = _attn_block(x2, r2(ln1a_s), r2(ln1a_b), qkv_a, proj_a_w, proj_a_b, n)
    x2 = _ffn(x2, r2(ln2a_s), r2(ln2a_b), ff_w1, r2(ff_b1), ff_w2, r2(ff_b2),
              n)

    x2 = _attn_block(x2, r2(ln1b_s), r2(ln1b_b), qkv_b, proj_b_w, proj_b_b, n)
    h, tw_t, ti_t = _ln_gate(x2, r2(ln2b_s), r2(ln2b_b), gate_w, n)

    pos0, pos1, te2 = _router(ti_t, n)
    te = te2.reshape(NTE)
    nrows = n * 2 + E * TR

    hh = _moe_up(te, h, pos0, pos1, e_w1, e_b1.reshape(E, 1, F),
                 e_w3, e_b3.reshape(E, 1, F), n, nrows)
    y = _moe_down(te, hh, e_w2, e_b2.reshape(E, 1, C), nrows)
    x2 = _combine(pos0.reshape(n), pos1.reshape(n), x2, tw_t, y, n, nrows)
    return x2.reshape(B, N, C)


# speedup vs baseline: 1.0178x; 1.0178x over previous
import functools

import jax
import jax.numpy as jnp
from jax.experimental import pallas as pl
from jax.experimental.pallas import tpu as pltpu
from jax.experimental.pallas import tpu_sc as plsc

H = 16
DH = 64
EPS = 1e-5
C = 1024
F = 2048
E = 8
TR = 256
RT = 256
NTE = 128
BF = jnp.bfloat16
F32 = jnp.float32


def _ln_f32(x, s, b):
    m = jnp.mean(x, axis=-1, keepdims=True)
    d = x - m
    v = jnp.mean(d * d, axis=-1, keepdims=True)
    return d * jax.lax.rsqrt(v + EPS) * s + b


def _dot_t(a, w):
    return jax.lax.dot_general(a, w, (((1,), (1,)), ((), ())),
                               preferred_element_type=F32)


def _k1_body(x_ref, s_ref, b_ref, w_ref, q_ref, k_ref, v_ref):
    h = _ln_f32(x_ref[...], s_ref[...], b_ref[...])
    qkv = _dot_t(h, w_ref[...]).astype(BF)
    for hd in range(H):
        q_ref[hd] = qkv[:, hd * DH:(hd + 1) * DH]
        k_ref[hd] = qkv[:, C + hd * DH:C + (hd + 1) * DH]
        v_ref[hd] = qkv[:, 2 * C + hd * DH:2 * C + (hd + 1) * DH]


def _ln_qkv(x, s, b, w, n):
    return pl.pallas_call(
        _k1_body,
        grid=(n // RT,),
        in_specs=[
            pl.BlockSpec((RT, C), lambda i: (i, 0)),
            pl.BlockSpec((1, C), lambda i: (0, 0)),
            pl.BlockSpec((1, C), lambda i: (0, 0)),
            pl.BlockSpec((3 * C, C), lambda i: (0, 0)),
        ],
        out_specs=[
            pl.BlockSpec((H, RT, DH), lambda i: (0, i, 0)),
            pl.BlockSpec((H, RT, DH), lambda i: (0, i, 0)),
            pl.BlockSpec((H, RT, DH), lambda i: (0, i, 0)),
        ],
        out_shape=[
            jax.ShapeDtypeStruct((H, n, DH), BF),
            jax.ShapeDtypeStruct((H, n, DH), BF),
            jax.ShapeDtypeStruct((H, n, DH), BF),
        ],
    )(x, s, b, w)


def _k2_body(q_ref, k_ref, v_ref, o_ref):
    q = (q_ref[0].astype(F32) * (DH ** -0.5)).astype(BF)
    k = k_ref[0]
    v = v_ref[0]
    s = jax.lax.dot_general(q, k, (((1,), (1,)), ((), ())),
                            preferred_element_type=F32).astype(BF)
    p = jnp.exp(s)
    l = jnp.sum(p.astype(F32), axis=-1, keepdims=True)
    o = jnp.dot(p, v, preferred_element_type=F32) / l
    o_ref[0] = o.astype(BF)


def _attention(q, k, v, n):
    return pl.pallas_call(
        _k2_body,
        grid=(H, n // RT),
        in_specs=[
            pl.BlockSpec((1, RT, DH), lambda h, i: (h, i, 0)),
            pl.BlockSpec((1, n, DH), lambda h, i: (h, 0, 0)),
            pl.BlockSpec((1, n, DH), lambda h, i: (h, 0, 0)),
        ],
        out_specs=pl.BlockSpec((1, RT, DH), lambda h, i: (h, i, 0)),
        out_shape=jax.ShapeDtypeStruct((H, n, DH), BF),
    )(q, k, v)


def _k3_body(x_ref, o_ref, w_ref, b_ref, out_ref):
    o = jnp.concatenate([o_ref[hd] for hd in range(H)], axis=1).astype(F32)
    out_ref[...] = x_ref[...] + _dot_t(o, w_ref[...]) + b_ref[...]


def _proj_res(x, o, w, b, n):
    return pl.pallas_call(
        _k3_body,
        grid=(n // RT,),
        in_specs=[
            pl.BlockSpec((RT, C), lambda i: (i, 0)),
            pl.BlockSpec((H, RT, DH), lambda i: (0, i, 0)),
            pl.BlockSpec((C, C), lambda i: (0, 0)),
            pl.BlockSpec((1, C), lambda i: (0, 0)),
        ],
        out_specs=pl.BlockSpec((RT, C), lambda i: (i, 0)),
        out_shape=jax.ShapeDtypeStruct((n, C), F32),
    )(x, o, w, b)


def _k4_body(x_ref, s_ref, b_ref, w1_ref, b1_ref, w2_ref, b2_ref, out_ref):
    x = x_ref[...]
    h = _ln_f32(x, s_ref[...], b_ref[...])
    a = _dot_t(h, w1_ref[...]) + b1_ref[...]
    g = a * 0.5 * (1.0 + jax.lax.erf(a * (2.0 ** -0.5)))
    out_ref[...] = x + _dot_t(g, w2_ref[...]) + b2_ref[...]


def _ffn(x, s, b, w1, b1, w2, b2, n):
    return pl.pallas_call(
        _k4_body,
        grid=(n // RT,),
        in_specs=[
            pl.BlockSpec((RT, C), lambda i: (i, 0)),
            pl.BlockSpec((1, C), lambda i: (0, 0)),
            pl.BlockSpec((1, C), lambda i: (0, 0)),
            pl.BlockSpec((F, C), lambda i: (0, 0)),
            pl.BlockSpec((1, F), lambda i: (0, 0)),
            pl.BlockSpec((C, F), lambda i: (0, 0)),
            pl.BlockSpec((1, C), lambda i: (0, 0)),
        ],
        out_specs=pl.BlockSpec((RT, C), lambda i: (i, 0)),
        out_shape=jax.ShapeDtypeStruct((n, C), F32),
    )(x, s, b, w1, b1, w2, b2)


def _k5_body(x_ref, s_ref, b_ref, gw_ref, h_ref, tw_ref, ti_ref):
    h = _ln_f32(x_ref[...], s_ref[...], b_ref[...])
    h_ref[...] = h
    sc = jax.lax.dot_general(h, gw_ref[...], (((1,), (1,)), ((), ())),
                             precision=jax.lax.Precision.HIGHEST,
                             preferred_element_type=F32)
    sc = sc - jnp.max(sc, axis=-1, keepdims=True)
    ex = jnp.exp(sc)
    p = ex / jnp.sum(ex, axis=-1, keepdims=True)
    idx = jax.lax.broadcasted_iota(jnp.int32, (RT, E), 1)
    m1 = jnp.max(p, axis=-1, keepdims=True)
    i1 = jnp.min(jnp.where(p == m1, idx, E), axis=-1, keepdims=True)
    pm = jnp.where(idx == i1, -1.0, p)
    m2 = jnp.max(pm, axis=-1, keepdims=True)
    i2 = jnp.min(jnp.where(pm == m2, idx, E), axis=-1, keepdims=True)
    tw_ref[...] = jnp.concatenate([m1, m2], axis=1)
    ti_ref[...] = jnp.concatenate([i1, i2], axis=1).T


def _ln_gate(x, s, b, gate_w, n):
    return pl.pallas_call(
        _k5_body,
        grid=(n // RT,),
        in_specs=[
            pl.BlockSpec((RT, C), lambda i: (i, 0)),
            pl.BlockSpec((1, C), lambda i: (0, 0)),
            pl.BlockSpec((1, C), lambda i: (0, 0)),
            pl.BlockSpec((E, C), lambda i: (0, 0)),
        ],
        out_specs=[
            pl.BlockSpec((RT, C), lambda i: (i, 0)),
            pl.BlockSpec((RT, 2), lambda i: (i, 0)),
            pl.BlockSpec((2, RT), lambda i: (0, i)),
        ],
        out_shape=[
            jax.ShapeDtypeStruct((n, C), F32),
            jax.ShapeDtypeStruct((n, 2), F32),
            jax.ShapeDtypeStruct((2, n), jnp.int32),
        ],
    )(x, s, b, gate_w)


def _shift_lanes(x, sh):
    z = jnp.zeros((x.shape[0], sh), x.dtype)
    return jnp.concatenate([z, x[:, :x.shape[1] - sh]], axis=1)


def _router_body(ti_ref, p0_ref, p1_ref, te_ref):
    n = ti_ref.shape[1]
    ti = ti_ref[...]
    masks = []
    counts = []
    for e in range(E):
        m = (ti == e).astype(jnp.int32)
        masks.append(m)
        counts.append(jnp.sum(m))
    r0 = jnp.zeros((1, n), jnp.int32)
    r1 = jnp.zeros((1, n), jnp.int32)
    te = jnp.zeros((1, NTE), jnp.int32)
    tile_base = jax.lax.broadcasted_iota(jnp.int32, (1, NTE), 1) * TR
    poff = jnp.zeros((), jnp.int32)
    for e in range(E):
        m = masks[e]
        pre = m
        sh = 1
        while sh < n:
            pre = pre + _shift_lanes(pre, sh)
            sh *= 2
        t0 = pre[0:1, n - 1:n]
        rank0 = pre[0:1] - 1
        rank1 = pre[1:2] + t0 - 1
        r0 = r0 + m[0:1] * (poff + rank0)
        r1 = r1 + m[1:2] * (poff + rank1)
        padded = ((counts[e] + TR - 1) // TR) * TR
        pend = poff + padded
        te = te + (tile_base >= pend).astype(jnp.int32)
        poff = pend
    p0_ref[...] = r0
    p1_ref[...] = r1
    te_ref[...] = jnp.minimum(te, E - 1)


def _router(ti_t, n):
    return pl.pallas_call(
        _router_body,
        grid=(1,),
        in_specs=[pl.BlockSpec((2, n), lambda i: (0, 0))],
        out_specs=[
            pl.BlockSpec((1, n), lambda i: (0, 0)),
            pl.BlockSpec((1, n), lambda i: (0, 0)),
            pl.BlockSpec((1, NTE), lambda i: (0, 0)),
        ],
        out_shape=[
            jax.ShapeDtypeStruct((1, n), jnp.int32),
            jax.ShapeDtypeStruct((1, n), jnp.int32),
            jax.ShapeDtypeStruct((1, NTE), jnp.int32),
        ],
    )(ti_t)


def _sc_dispatch(h, p0, p1, n, nrows):
    info = plsc.get_sparse_core_info()
    nc, ns = info.num_cores, info.num_subcores
    bpw = n // (nc * ns)
    mesh = plsc.VectorSubcoreMesh(core_axis_name="c", subcore_axis_name="s")

    @functools.partial(
        pl.kernel, mesh=mesh,
        out_type=jax.ShapeDtypeStruct((nrows, C), F32),
        scratch_types=[
            pltpu.VMEM((bpw,), jnp.int32),
            pltpu.VMEM((bpw, C), F32),
            pltpu.SemaphoreType.DMA,
        ],
    )
    def k(h_hbm, p0_hbm, p1_hbm, xg_hbm, idx_v, rows_v, sem):
        wid = jax.lax.axis_index("s") * nc + jax.lax.axis_index("c")
        base = wid * bpw
        pltpu.sync_copy(h_hbm.at[pl.ds(base, bpw)], rows_v)
        pltpu.sync_copy(p0_hbm.at[pl.ds(base, bpw)], idx_v)
        pltpu.async_copy(rows_v, xg_hbm.at[idx_v], sem).wait()
        pltpu.sync_copy(p1_hbm.at[pl.ds(base, bpw)], idx_v)
        pltpu.async_copy(rows_v, xg_hbm.at[idx_v], sem).wait()

    return k(h, p0, p1)


def _k6a_body(te_ref, xg_ref, w1_ref, b1_ref, w3_ref, b3_ref, hh_ref):
    xg = xg_ref[...]
    a = _dot_t(xg, w1_ref[0]) + b1_ref[0]
    g = a * jax.nn.sigmoid(a)
    c = _dot_t(xg, w3_ref[0]) + b3_ref[0]
    hh_ref[...] = (g * c).astype(BF)


def _moe_up(te, xg, w1, b1, w3, b3, n, nrows):
    grid_spec = pltpu.PrefetchScalarGridSpec(
        num_scalar_prefetch=1,
        grid=(nrows // TR,),
        in_specs=[
            pl.BlockSpec((TR, C), lambda t, te_ref: (t, 0)),
            pl.BlockSpec((1, F, C), lambda t, te_ref: (te_ref[t], 0, 0)),
            pl.BlockSpec((1, 1, F), lambda t, te_ref: (te_ref[t], 0, 0)),
            pl.BlockSpec((1, F, C), lambda t, te_ref: (te_ref[t], 0, 0)),
            pl.BlockSpec((1, 1, F), lambda t, te_ref: (te_ref[t], 0, 0)),
        ],
        out_specs=pl.BlockSpec((TR, F), lambda t, te_ref: (t, 0)),
    )
    return pl.pallas_call(
        _k6a_body,
        grid_spec=grid_spec,
        out_shape=jax.ShapeDtypeStruct((nrows, F), BF),
    )(te, xg, w1, b1, w3, b3)


def _k6b_body(te_ref, hh_ref, w2_ref, b2_ref, y_ref):
    hh = hh_ref[...].astype(F32)
    y_ref[...] = _dot_t(hh, w2_ref[0]) + b2_ref[0]


def _moe_down(te, hh, w2, b2, nrows):
    grid_spec = pltpu.PrefetchScalarGridSpec(
        num_scalar_prefetch=1,
        grid=(nrows // TR,),
        in_specs=[
            pl.BlockSpec((TR, F), lambda t, te_ref: (t, 0)),
            pl.BlockSpec((1, C, F), lambda t, te_ref: (te_ref[t], 0, 0)),
            pl.BlockSpec((1, 1, C), lambda t, te_ref: (te_ref[t], 0, 0)),
        ],
        out_specs=pl.BlockSpec((TR, C), lambda t, te_ref: (t, 0)),
    )
    return pl.pallas_call(
        _k6b_body,
        grid_spec=grid_spec,
        out_shape=jax.ShapeDtypeStruct((nrows, C), F32),
    )(te, hh, w2, b2)


def _sc_gather(y, p0, p1, n, nrows):
    info = plsc.get_sparse_core_info()
    nc, ns = info.num_cores, info.num_subcores
    bpw = n // (nc * ns)
    mesh = plsc.VectorSubcoreMesh(core_axis_name="c", subcore_axis_name="s")

    @functools.partial(
        pl.kernel, mesh=mesh,
        out_type=jax.ShapeDtypeStruct((2, n, C), F32),
        scratch_types=[
            pltpu.VMEM((bpw,), jnp.int32),
            pltpu.VMEM((bpw, C), F32),
            pltpu.SemaphoreType.DMA,
        ],
    )
    def k(y_hbm, p0_hbm, p1_hbm, out_hbm, idx_v, rows_v, sem):
        wid = jax.lax.axis_index("s") * nc + jax.lax.axis_index("c")
        base = wid * bpw
        pltpu.sync_copy(p0_hbm.at[pl.ds(base, bpw)], idx_v)
        pltpu.async_copy(y_hbm.at[idx_v], rows_v, sem).wait()
        pltpu.sync_copy(rows_v, out_hbm.at[0, pl.ds(base, bpw)])
        pltpu.sync_copy(p1_hbm.at[pl.ds(base, bpw)], idx_v)
        pltpu.async_copy(y_hbm.at[idx_v], rows_v, sem).wait()
        pltpu.sync_copy(rows_v, out_hbm.at[1, pl.ds(base, bpw)])

    return k(y, p0, p1)


def _k7_body(x_ref, tw_ref, g_ref, out_ref):
    out_ref[...] = (x_ref[...]
                    + tw_ref[:, 0:1] * g_ref[0]
                    + tw_ref[:, 1:2] * g_ref[1])


def _combine(pos0, pos1, x, tw, y, n, nrows):
    yg = _sc_gather(y, pos0, pos1, n, nrows)
    return pl.pallas_call(
        _k7_body,
        grid=(n // RT,),
        in_specs=[
            pl.BlockSpec((RT, C), lambda t: (t, 0)),
            pl.BlockSpec((RT, 2), lambda t: (t, 0)),
            pl.BlockSpec((2, RT, C), lambda t: (0, t, 0)),
        ],
        out_specs=pl.BlockSpec((RT, C), lambda t: (t, 0)),
        out_shape=jax.ShapeDtypeStruct((n, C), F32),
    )(x, tw, yg)


def _attn_block(x, ln_s, ln_b, qkv_w, proj_w, proj_b, n):
    q, k, v = _ln_qkv(x, ln_s, ln_b, qkv_w, n)
    o = _attention(q, k, v, n)
    return _proj_res(x, o, proj_w, proj_b.reshape(1, C), n)


def kernel(x, ln1a_s, ln1a_b, qkv_a, proj_a_w, proj_a_b, ln2a_s, ln2a_b,
           ff_w1, ff_b1, ff_w2, ff_b2, ln1b_s, ln1b_b, qkv_b, proj_b_w,
           proj_b_b, ln2b_s, ln2b_b, gate_w, e_w1, e_b1, e_w2, e_b2, e_w3,
           e_b3):
    B, N, _ = x.shape
    n = B * N
    x2 = x.reshape(n, C)
    r2 = lambda a: a.reshape(1, -1)

    x2 = _attn_block(x2, r2(ln1a_s), r2(ln1a_b), qkv_a, proj_a_w, proj_a_b, n)
    x2 = _ffn(x2, r2(ln2a_s), r2(ln2a_b), ff_w1, r2(ff_b1), ff_w2, r2(ff_b2),
              n)

    x2 = _attn_block(x2, r2(ln1b_s), r2(ln1b_b), qkv_b, proj_b_w, proj_b_b, n)
    h, tw_t, ti_t = _ln_gate(x2, r2(ln2b_s), r2(ln2b_b), gate_w, n)

    pos0, pos1, te2 = _router(ti_t, n)
    te = te2.reshape(NTE)
    nrows = n * 2 + E * TR

    xg = _sc_dispatch(h, pos0.reshape(n), pos1.reshape(n), n, nrows)
    hh = _moe_up(te, xg, e_w1, e_b1.reshape(E, 1, F),
                 e_w3, e_b3.reshape(E, 1, F), n, nrows)
    y = _moe_down(te, hh, e_w2, e_b2.reshape(E, 1, C), nrows)
    x2 = _combine(pos0.reshape(n), pos1.reshape(n), x2, tw_t, y, n, nrows)
    return x2.reshape(B, N, C)
